# initial kernel scaffold (unmeasured)
import functools

import jax
import jax.numpy as jnp
from jax import lax
from jax.experimental import pallas as pl
from jax.experimental.pallas import tpu as pltpu

N_DEV = 4

_sem_signal = getattr(pl, "semaphore_signal", None) or pltpu.semaphore_signal
_sem_wait = getattr(pl, "semaphore_wait", None) or pltpu.semaphore_wait


def _partial_body(x_ref, wg_ref, wu_ref, wd_ref, out_ref):
    j = pl.program_id(1)
    g = jnp.dot(x_ref[:, :], wg_ref[:, :], preferred_element_type=jnp.float32)
    u = jnp.dot(x_ref[:, :], wu_ref[:, :], preferred_element_type=jnp.float32)
    h = g * (u * jax.nn.sigmoid(u))
    p = jnp.dot(h, wd_ref[:, :], preferred_element_type=jnp.float32)

    @pl.when(j == 0)
    def _():
        out_ref[:, :] = p

    @pl.when(j != 0)
    def _():
        out_ref[:, :] += p


def _compute_partial(x, Wg, Wu, Wd):
    M, K = x.shape
    H = Wg.shape[1]
    D = Wd.shape[1]
    BM, BH = 1024, 512
    grid = (M // BM, H // BH)
    return pl.pallas_call(
        _partial_body,
        grid=grid,
        in_specs=[
            pl.BlockSpec((BM, K), lambda i, j: (i, 0)),
            pl.BlockSpec((K, BH), lambda i, j: (0, j)),
            pl.BlockSpec((K, BH), lambda i, j: (0, j)),
            pl.BlockSpec((BH, D), lambda i, j: (j, 0)),
        ],
        out_specs=pl.BlockSpec((BM, D), lambda i, j: (i, 0)),
        out_shape=jax.ShapeDtypeStruct((M, D), jnp.float32),
        compiler_params=pltpu.CompilerParams(
            dimension_semantics=("parallel", "arbitrary"),
        ),
    )(x, Wg, Wu, Wd)


def _allreduce_body(p_ref, out_ref, recv_buf, rs_send, rs_recv, ag_send, ag_recv):
    my = lax.axis_index("i")
    left = (my + N_DEV - 1) % N_DEV
    right = (my + 1) % N_DEV
    M, D = out_ref.shape
    CH = M // N_DEV

    barrier_sem = pltpu.get_barrier_semaphore()
    for nbr in (left, right):
        _sem_signal(
            barrier_sem, inc=1,
            device_id=(nbr,), device_id_type=pl.DeviceIdType.MESH,
        )
    _sem_wait(barrier_sem, 2)

    out_ref[:, :] = p_ref[:, :]

    for s in range(N_DEV - 1):
        send_idx = (my + N_DEV - s) % N_DEV
        recv_idx = (my + N_DEV - s - 1) % N_DEV
        rdma = pltpu.make_async_remote_copy(
            src_ref=out_ref.at[pl.ds(send_idx * CH, CH), :],
            dst_ref=recv_buf.at[s],
            send_sem=rs_send.at[s],
            recv_sem=rs_recv.at[s],
            device_id=(right,),
            device_id_type=pl.DeviceIdType.MESH,
        )
        rdma.start()
        rdma.wait()
        out_ref[pl.ds(recv_idx * CH, CH), :] += recv_buf[s]

    for t in range(N_DEV - 1):
        src_idx = (my + N_DEV + 1 - t) % N_DEV
        sl = pl.ds(src_idx * CH, CH)
        rdma = pltpu.make_async_remote_copy(
            src_ref=out_ref.at[sl, :],
            dst_ref=out_ref.at[sl, :],
            send_sem=ag_send.at[t],
            recv_sem=ag_recv.at[t],
            device_id=(right,),
            device_id_type=pl.DeviceIdType.MESH,
        )
        rdma.start()
        rdma.wait()


def _allreduce(partial):
    M, D = partial.shape
    CH = M // N_DEV
    return pl.pallas_call(
        _allreduce_body,
        in_specs=[pl.BlockSpec(memory_space=pltpu.VMEM)],
        out_specs=pl.BlockSpec(memory_space=pltpu.VMEM),
        out_shape=jax.ShapeDtypeStruct((M, D), jnp.float32),
        scratch_shapes=[
            pltpu.VMEM((N_DEV - 1, CH, D), jnp.float32),
            pltpu.SemaphoreType.DMA((N_DEV - 1,)),
            pltpu.SemaphoreType.DMA((N_DEV - 1,)),
            pltpu.SemaphoreType.DMA((N_DEV - 1,)),
            pltpu.SemaphoreType.DMA((N_DEV - 1,)),
        ],
        compiler_params=pltpu.CompilerParams(collective_id=0),
    )(partial)


def kernel(x, Wg, Wu, Wd):
    partial = _compute_partial(x, Wg, Wu, Wd)
    return _allreduce(partial)


# baseline (device time: 448251 ns/iter reference)
import functools

import jax
import jax.numpy as jnp
from jax import lax
from jax.experimental import pallas as pl
from jax.experimental.pallas import tpu as pltpu

N_DEV = 4

_sem_signal = getattr(pl, "semaphore_signal", None) or pltpu.semaphore_signal
_sem_wait = getattr(pl, "semaphore_wait", None) or pltpu.semaphore_wait


def _partial_body(x_ref, wg_ref, wu_ref, wd_ref, out_ref):
    j = pl.program_id(1)
    g = jnp.dot(x_ref[:, :], wg_ref[:, :], preferred_element_type=jnp.float32)
    u = jnp.dot(x_ref[:, :], wu_ref[:, :], preferred_element_type=jnp.float32)
    h = g * (u * jax.nn.sigmoid(u))
    p = jnp.dot(h, wd_ref[:, :], preferred_element_type=jnp.float32)

    @pl.when(j == 0)
    def _():
        out_ref[:, :] = p

    @pl.when(j != 0)
    def _():
        out_ref[:, :] += p


def _compute_partial(x, Wg, Wu, Wd):
    M, K = x.shape
    H = Wg.shape[1]
    D = Wd.shape[1]
    BM, BH = 1024, 512
    grid = (M // BM, H // BH)
    return pl.pallas_call(
        _partial_body,
        grid=grid,
        in_specs=[
            pl.BlockSpec((BM, K), lambda i, j: (i, 0)),
            pl.BlockSpec((K, BH), lambda i, j: (0, j)),
            pl.BlockSpec((K, BH), lambda i, j: (0, j)),
            pl.BlockSpec((BH, D), lambda i, j: (j, 0)),
        ],
        out_specs=pl.BlockSpec((BM, D), lambda i, j: (i, 0)),
        out_shape=jax.ShapeDtypeStruct((M, D), jnp.float32),
        compiler_params=pltpu.CompilerParams(
            dimension_semantics=("parallel", "arbitrary"),
            vmem_limit_bytes=100 * 1024 * 1024,
        ),
    )(x, Wg, Wu, Wd)


def _allreduce_body(p_ref, out_ref, recv_buf, rs_send, rs_recv, ag_send, ag_recv):
    my = lax.axis_index("i")
    left = (my + N_DEV - 1) % N_DEV
    right = (my + 1) % N_DEV
    M, D = out_ref.shape
    CH = M // N_DEV

    barrier_sem = pltpu.get_barrier_semaphore()
    for nbr in (left, right):
        _sem_signal(
            barrier_sem, inc=1,
            device_id=(nbr,), device_id_type=pl.DeviceIdType.MESH,
        )
    _sem_wait(barrier_sem, 2)

    out_ref[:, :] = p_ref[:, :]

    for s in range(N_DEV - 1):
        send_idx = (my + N_DEV - s) % N_DEV
        recv_idx = (my + N_DEV - s - 1) % N_DEV
        rdma = pltpu.make_async_remote_copy(
            src_ref=out_ref.at[pl.ds(send_idx * CH, CH), :],
            dst_ref=recv_buf.at[s],
            send_sem=rs_send.at[s],
            recv_sem=rs_recv.at[s],
            device_id=(right,),
            device_id_type=pl.DeviceIdType.MESH,
        )
        rdma.start()
        rdma.wait()
        out_ref[pl.ds(recv_idx * CH, CH), :] += recv_buf[s]

    for t in range(N_DEV - 1):
        src_idx = (my + N_DEV + 1 - t) % N_DEV
        sl = pl.ds(src_idx * CH, CH)
        rdma = pltpu.make_async_remote_copy(
            src_ref=out_ref.at[sl, :],
            dst_ref=out_ref.at[sl, :],
            send_sem=ag_send.at[t],
            recv_sem=ag_recv.at[t],
            device_id=(right,),
            device_id_type=pl.DeviceIdType.MESH,
        )
        rdma.start()
        rdma.wait()


def _allreduce(partial):
    M, D = partial.shape
    CH = M // N_DEV
    return pl.pallas_call(
        _allreduce_body,
        in_specs=[pl.BlockSpec(memory_space=pltpu.VMEM)],
        out_specs=pl.BlockSpec(memory_space=pltpu.VMEM),
        out_shape=jax.ShapeDtypeStruct((M, D), jnp.float32),
        scratch_shapes=[
            pltpu.VMEM((N_DEV - 1, CH, D), jnp.float32),
            pltpu.SemaphoreType.DMA((N_DEV - 1,)),
            pltpu.SemaphoreType.DMA((N_DEV - 1,)),
            pltpu.SemaphoreType.DMA((N_DEV - 1,)),
            pltpu.SemaphoreType.DMA((N_DEV - 1,)),
        ],
        compiler_params=pltpu.CompilerParams(
            collective_id=0,
            vmem_limit_bytes=100 * 1024 * 1024,
        ),
    )(partial)


def kernel(x, Wg, Wu, Wd):
    partial = _compute_partial(x, Wg, Wu, Wd)
    return _allreduce(partial)


# device time: 247666 ns/iter; 1.8099x vs baseline; 1.8099x over previous
import jax
import jax.numpy as jnp
from jax import lax
from jax.experimental import pallas as pl
from jax.experimental.pallas import tpu as pltpu

N_DEV = 4
M, D = 2048, 2048
BM = M // N_DEV
HALF = BM // 2
SUB = HALF // N_DEV
BH = 512
N_H = 8
N_EPOCH = N_DEV + 2
SPACING = 2

_sem_signal = getattr(pl, "semaphore_signal", None) or pltpu.semaphore_signal
_sem_wait = getattr(pl, "semaphore_wait", None) or pltpu.semaphore_wait


def _fused_body(x_ref, wg_ref, wu_ref, wd_ref, out_ref,
                recv_buf, rs_send, rs_recv, ag_send, ag_recv):
    i = pl.program_id(0)
    j = pl.program_id(1)
    ts = i * N_H + j
    my = lax.axis_index("i")
    left = (my + N_DEV - 1) % N_DEV
    right = (my + 1) % N_DEV

    @pl.when(ts == 0)
    def _():
        barrier_sem = pltpu.get_barrier_semaphore()
        for nbr in (left, right):
            _sem_signal(
                barrier_sem, inc=1,
                device_id=(nbr,), device_id_type=pl.DeviceIdType.MESH,
            )
        _sem_wait(barrier_sem, 2)

    def rows(g, dirn, k):
        return pl.ds(g * BM + dirn * HALF + k * SUB, SUB)

    def rs_send_idx(dirn, s):
        return (my + (N_DEV - s if dirn == 0 else s)) % N_DEV

    def rs_recv_idx(dirn, s):
        return (my + (N_DEV - s - 1 if dirn == 0 else s + 1)) % N_DEV

    def ag_send_idx(dirn, t):
        return (my + (N_DEV + 1 - t if dirn == 0 else N_DEV - 1 + t)) % N_DEV

    def tgt(dirn):
        return right if dirn == 0 else left

    def rs_desc(g, dirn, s):
        return pltpu.make_async_remote_copy(
            src_ref=out_ref.at[rows(g, dirn, rs_send_idx(dirn, s)), :],
            dst_ref=recv_buf.at[g, dirn, s],
            send_sem=rs_send.at[g, dirn, s],
            recv_sem=rs_recv.at[g, dirn, s],
            device_id=(tgt(dirn),),
            device_id_type=pl.DeviceIdType.MESH,
        )

    def ag_desc(g, dirn, t):
        sl = rows(g, dirn, ag_send_idx(dirn, t))
        return pltpu.make_async_remote_copy(
            src_ref=out_ref.at[sl, :],
            dst_ref=out_ref.at[sl, :],
            send_sem=ag_send.at[g, dirn, t],
            recv_sem=ag_recv.at[g, dirn, t],
            device_id=(tgt(dirn),),
            device_id_type=pl.DeviceIdType.MESH,
        )

    def hop(g, k):
        for dirn in (0, 1):
            if k == 0:
                rs_desc(g, dirn, 0).start()
            elif k in (1, 2):
                s = k - 1
                rs_desc(g, dirn, s).wait_recv()
                out_ref[rows(g, dirn, rs_recv_idx(dirn, s)), :] += \
                    recv_buf[g, dirn, s, :, :]
                rs_desc(g, dirn, k).start()
            elif k == 3:
                rs_desc(g, dirn, 2).wait_recv()
                out_ref[rows(g, dirn, rs_recv_idx(dirn, 2)), :] += \
                    recv_buf[g, dirn, 2, :, :]
                ag_desc(g, dirn, 0).start()
            elif k in (4, 5):
                t = k - 4
                ag_desc(g, dirn, t).wait_recv()
                ag_desc(g, dirn, t + 1).start()
            else:
                ag_desc(g, dirn, 2).wait_recv()
                for s in range(N_DEV - 1):
                    rs_desc(g, dirn, s).wait_send()
                    ag_desc(g, dirn, s).wait_send()

    for g in range(N_DEV):
        for k in range(7):
            @pl.when(ts == (g + 1) * N_H + k * SPACING)
            def _(g=g, k=k):
                hop(g, k)

    @pl.when(i < N_DEV)
    def _():
        gate = jnp.dot(x_ref[:, :], wg_ref[:, :],
                       preferred_element_type=jnp.float32)
        up = jnp.dot(x_ref[:, :], wu_ref[:, :],
                     preferred_element_type=jnp.float32)
        h = gate * (up * jax.nn.sigmoid(up))
        p = jnp.dot(h, wd_ref[:, :], preferred_element_type=jnp.float32)
        sl = pl.ds(i * BM, BM)

        @pl.when(j == 0)
        def _():
            out_ref[sl, :] = p

        @pl.when(j != 0)
        def _():
            out_ref[sl, :] += p


def _wblk_idx(i, j):
    return jnp.where(i >= N_DEV, N_H - 1, j)


def kernel(x, Wg, Wu, Wd):
    K = x.shape[1]
    n_dirs, n_steps = 2, N_DEV - 1
    return pl.pallas_call(
        _fused_body,
        grid=(N_EPOCH, N_H),
        in_specs=[
            pl.BlockSpec((BM, K), lambda i, j: (jnp.minimum(i, N_DEV - 1), 0)),
            pl.BlockSpec((K, BH), lambda i, j: (0, _wblk_idx(i, j))),
            pl.BlockSpec((K, BH), lambda i, j: (0, _wblk_idx(i, j))),
            pl.BlockSpec((BH, D), lambda i, j: (_wblk_idx(i, j), 0)),
        ],
        out_specs=pl.BlockSpec((M, D), lambda i, j: (0, 0)),
        out_shape=jax.ShapeDtypeStruct((M, D), jnp.float32),
        scratch_shapes=[
            pltpu.VMEM((N_DEV, n_dirs, n_steps, SUB, D), jnp.float32),
            pltpu.SemaphoreType.DMA((N_DEV, n_dirs, n_steps)),
            pltpu.SemaphoreType.DMA((N_DEV, n_dirs, n_steps)),
            pltpu.SemaphoreType.DMA((N_DEV, n_dirs, n_steps)),
            pltpu.SemaphoreType.DMA((N_DEV, n_dirs, n_steps)),
        ],
        compiler_params=pltpu.CompilerParams(
            dimension_semantics=("arbitrary", "arbitrary"),
            collective_id=0,
            vmem_limit_bytes=100 * 1024 * 1024,
        ),
    )(x, Wg, Wu, Wd)


# device time: 243977 ns/iter; 1.8373x vs baseline; 1.0151x over previous
import jax
import jax.numpy as jnp
from jax import lax
from jax.experimental import pallas as pl
from jax.experimental.pallas import tpu as pltpu

N_DEV = 4
M, D = 2048, 2048
BM = M // N_DEV
HALF = BM // 2
SUB = HALF // N_DEV
BH = 512
N_H = 8
N_EPOCH = N_DEV + 2
SPACING = 3


def _hop_step(g, k):
    if g == N_DEV - 1:
        return N_DEV * N_H + k
    return (g + 1) * N_H + k * SPACING

_sem_signal = getattr(pl, "semaphore_signal", None) or pltpu.semaphore_signal
_sem_wait = getattr(pl, "semaphore_wait", None) or pltpu.semaphore_wait


def _fused_body(x_ref, wg_ref, wu_ref, wd_ref, out_ref,
                acc_ref, recv_buf, rs_send, rs_recv, ag_send, ag_recv):
    i = pl.program_id(0)
    j = pl.program_id(1)
    ts = i * N_H + j
    my = lax.axis_index("i")
    left = (my + N_DEV - 1) % N_DEV
    right = (my + 1) % N_DEV

    @pl.when(ts == 0)
    def _():
        barrier_sem = pltpu.get_barrier_semaphore()
        for nbr in (left, right):
            _sem_signal(
                barrier_sem, inc=1,
                device_id=(nbr,), device_id_type=pl.DeviceIdType.MESH,
            )
        _sem_wait(barrier_sem, 2)

    def rows(g, dirn, k):
        return pl.ds(g * BM + dirn * HALF + k * SUB, SUB)

    def rs_send_idx(dirn, s):
        return (my + (N_DEV - s if dirn == 0 else s)) % N_DEV

    def rs_recv_idx(dirn, s):
        return (my + (N_DEV - s - 1 if dirn == 0 else s + 1)) % N_DEV

    def ag_send_idx(dirn, t):
        return (my + (N_DEV + 1 - t if dirn == 0 else N_DEV - 1 + t)) % N_DEV

    def tgt(dirn):
        return right if dirn == 0 else left

    def rs_desc(g, dirn, s):
        return pltpu.make_async_remote_copy(
            src_ref=out_ref.at[rows(g, dirn, rs_send_idx(dirn, s)), :],
            dst_ref=recv_buf.at[g % 2, dirn, s],
            send_sem=rs_send.at[g, dirn, s],
            recv_sem=rs_recv.at[g, dirn, s],
            device_id=(tgt(dirn),),
            device_id_type=pl.DeviceIdType.MESH,
        )

    def ag_desc(g, dirn, t):
        sl = rows(g, dirn, ag_send_idx(dirn, t))
        return pltpu.make_async_remote_copy(
            src_ref=out_ref.at[sl, :],
            dst_ref=out_ref.at[sl, :],
            send_sem=ag_send.at[g, dirn, t],
            recv_sem=ag_recv.at[g, dirn, t],
            device_id=(tgt(dirn),),
            device_id_type=pl.DeviceIdType.MESH,
        )

    def hop(g, k):
        for dirn in (0, 1):
            if k == 0:
                rs_desc(g, dirn, 0).start()
            elif k in (1, 2):
                s = k - 1
                rs_desc(g, dirn, s).wait_recv()
                out_ref[rows(g, dirn, rs_recv_idx(dirn, s)), :] += \
                    recv_buf[g % 2, dirn, s, :, :]
                rs_desc(g, dirn, k).start()
            elif k == 3:
                rs_desc(g, dirn, 2).wait_recv()
                out_ref[rows(g, dirn, rs_recv_idx(dirn, 2)), :] += \
                    recv_buf[g % 2, dirn, 2, :, :]
                ag_desc(g, dirn, 0).start()
            elif k in (4, 5):
                t = k - 4
                ag_desc(g, dirn, t).wait_recv()
                ag_desc(g, dirn, t + 1).start()
            else:
                ag_desc(g, dirn, 2).wait_recv()
                for s in range(N_DEV - 1):
                    rs_desc(g, dirn, s).wait_send()
                    ag_desc(g, dirn, s).wait_send()

    for g in range(N_DEV):
        for k in range(7):
            @pl.when(ts == _hop_step(g, k))
            def _(g=g, k=k):
                hop(g, k)

    @pl.when(i < N_DEV)
    def _():
        gate = jnp.dot(x_ref[:, :], wg_ref[:, :],
                       preferred_element_type=jnp.float32)
        up = jnp.dot(x_ref[:, :], wu_ref[:, :],
                     preferred_element_type=jnp.float32)
        h = gate * (up * jax.nn.sigmoid(up))
        p = jnp.dot(h, wd_ref[:, :], preferred_element_type=jnp.float32)

        @pl.when(j == 0)
        def _():
            acc_ref[:, :] = p

        @pl.when(j != 0)
        def _():
            acc_ref[:, :] += p

        @pl.when(j == N_H - 1)
        def _():
            out_ref[pl.ds(i * BM, BM), :] = acc_ref[:, :]


def _wblk_idx(i, j):
    return jnp.where(i >= N_DEV, N_H - 1, j)


def kernel(x, Wg, Wu, Wd):
    K = x.shape[1]
    n_dirs, n_steps = 2, N_DEV - 1
    return pl.pallas_call(
        _fused_body,
        grid=(N_EPOCH, N_H),
        in_specs=[
            pl.BlockSpec((BM, K), lambda i, j: (jnp.minimum(i, N_DEV - 1), 0)),
            pl.BlockSpec((K, BH), lambda i, j: (0, _wblk_idx(i, j))),
            pl.BlockSpec((K, BH), lambda i, j: (0, _wblk_idx(i, j))),
            pl.BlockSpec((BH, D), lambda i, j: (_wblk_idx(i, j), 0)),
        ],
        out_specs=pl.BlockSpec((M, D), lambda i, j: (0, 0)),
        out_shape=jax.ShapeDtypeStruct((M, D), jnp.float32),
        scratch_shapes=[
            pltpu.VMEM((BM, D), jnp.float32),
            pltpu.VMEM((2, n_dirs, n_steps, SUB, D), jnp.float32),
            pltpu.SemaphoreType.DMA((N_DEV, n_dirs, n_steps)),
            pltpu.SemaphoreType.DMA((N_DEV, n_dirs, n_steps)),
            pltpu.SemaphoreType.DMA((N_DEV, n_dirs, n_steps)),
            pltpu.SemaphoreType.DMA((N_DEV, n_dirs, n_steps)),
        ],
        compiler_params=pltpu.CompilerParams(
            dimension_semantics=("arbitrary", "arbitrary"),
            collective_id=0,
            vmem_limit_bytes=100 * 1024 * 1024,
        ),
    )(x, Wg, Wu, Wd)
